# baseline (speedup 1.0000x reference)
import functools

import jax
import jax.numpy as jnp
from jax import lax
from jax.experimental import pallas as pl
from jax.experimental.pallas import tpu as pltpu
from jax.experimental.pallas import tpu_sc as plsc

L = 16
NC = 2
NS = 16
NW = NC * NS
R = 2

CE = 2000
KB = 64
NEG_INF = float("-inf")


def _seg_agg_kernel(n_pad, e_total, d):
    n_half = n_pad // R
    n_w = n_half // NW
    n_core = n_w * NS
    n_chunks = e_total // CE
    dv = d // L

    mesh = plsc.VectorSubcoreMesh(core_axis_name="c", subcore_axis_name="s")

    @functools.partial(
        pl.kernel,
        out_type=(
            jax.ShapeDtypeStruct((n_pad, d), jnp.float32),
            jax.ShapeDtypeStruct((n_pad, d), jnp.float32),
        ),
        mesh=mesh,
        compiler_params=pltpu.CompilerParams(needs_layout_passes=False),
        scratch_types=dict(
            erow=pltpu.VMEM((CE,), jnp.int32),
            ecol=pltpu.VMEM((CE,), jnp.int32),
            crow=pltpu.VMEM((CE + KB,), jnp.int32),
            cmax=pltpu.VMEM((CE + KB,), jnp.int32),
            rbuf=pltpu.VMEM((KB, d), jnp.float32),
            amax=pltpu.VMEM((n_w + 1, d), jnp.float32),
            asum=pltpu.VMEM((n_w + 1, d), jnp.float32),
            sem=pltpu.SemaphoreType.DMA,
        ),
    )
    def seg_agg(row_hbm, col_hbm, x_hbm, out_sum, out_max,
                erow, ecol, crow, cmax, rbuf, amax, asum, sem):
        c = lax.axis_index("c")
        s = lax.axis_index("s")
        zeros = jnp.zeros((L,), jnp.float32)
        ninf = jnp.full((L,), NEG_INF, jnp.float32)
        lane = jnp.arange(L, dtype=jnp.int32)
        trash = CE + KB - L

        def pass_body(r, _):
            lo = r * n_half + c * n_core + s * n_w

            def init_row(i, _):
                for t in range(dv):
                    amax[i, pl.ds(t * L, L)] = ninf
                    asum[i, pl.ds(t * L, L)] = zeros
                return 0
            lax.fori_loop(0, n_w + 1, init_row, 0)

            def chunk_body(ci, _):
                pltpu.sync_copy(row_hbm.at[pl.ds(ci * CE, CE)], erow)
                pltpu.sync_copy(col_hbm.at[pl.ds(ci * CE, CE)], ecol)

                def filt(g, m):
                    col = ecol[pl.ds(g * L, L)]
                    rel = col - lo
                    mask = (rel >= 0) & (rel < n_w)
                    row = erow[pl.ds(g * L, L)]
                    cnt = jnp.where(mask, 1, 0)
                    for k in (1, 2, 4, 8):
                        sh = cnt.at[jnp.maximum(lane - k, 0)].get(
                            mode='promise_in_bounds')
                        cnt = cnt + jnp.where(lane >= k, sh, 0)
                    slot = jnp.where(mask, m + cnt - 1, trash + lane)
                    plsc.store_scatter(crow, [slot], row)
                    plsc.store_scatter(cmax, [slot], rel)
                    return m + cnt[L - 1]
                m = lax.fori_loop(0, CE // L, filt, 0)

                zi = jnp.zeros((L,), jnp.int32)
                ti = jnp.full((L,), n_w, jnp.int32)
                for t in range(KB // L):
                    crow[pl.ds(m + t * L, L)] = zi
                    cmax[pl.ds(m + t * L, L)] = ti

                nb = (m + KB - 1) // KB

                def batch(bi, _):
                    k0 = bi * KB
                    pltpu.async_copy(
                        x_hbm.at[crow.at[pl.ds(k0, KB)]], rbuf, sem).wait()
                    def edge(j, _):
                        rr = cmax[pl.ds(k0 + j, L)][0]
                        for t in range(dv):
                            sl = pl.ds(t * L, L)
                            v = rbuf[j, sl]
                            amax[rr, sl] = jnp.maximum(amax[rr, sl], v)
                            asum[rr, sl] = asum[rr, sl] + v
                        return 0
                    lax.fori_loop(0, KB, edge, 0)
                    return 0
                lax.fori_loop(0, nb, batch, 0)
                return 0
            lax.fori_loop(0, n_chunks, chunk_body, 0)

            def fix_row(i, _):
                for t in range(dv):
                    sl = pl.ds(t * L, L)
                    v = amax[i, sl]
                    amax[i, sl] = jnp.where(v == NEG_INF, 0.0, v)
                return 0
            lax.fori_loop(0, n_w, fix_row, 0)

            pltpu.sync_copy(amax.at[pl.ds(0, n_w)],
                            out_max.at[pl.ds(lo, n_w)])
            pltpu.sync_copy(asum.at[pl.ds(0, n_w)],
                            out_sum.at[pl.ds(lo, n_w)])
            return 0
        lax.fori_loop(0, R, pass_body, 0)

    return seg_agg


def _merge_kernel(n_pad, d):
    blk = 512
    grid = n_pad // blk

    def body(x_ref, s_ref, m_ref, w1_ref, w2_ref, b_ref, o_ref):
        acc = jnp.dot(s_ref[...], w1_ref[...],
                      preferred_element_type=jnp.float32)
        acc += jnp.dot(m_ref[...], w2_ref[...],
                       preferred_element_type=jnp.float32)
        o_ref[...] = acc + x_ref[...] + b_ref[...]

    return pl.pallas_call(
        body,
        grid=(grid,),
        in_specs=[
            pl.BlockSpec((blk, d), lambda i: (i, 0)),
            pl.BlockSpec((blk, d), lambda i: (i, 0)),
            pl.BlockSpec((blk, d), lambda i: (i, 0)),
            pl.BlockSpec((d, d), lambda i: (0, 0)),
            pl.BlockSpec((d, d), lambda i: (0, 0)),
            pl.BlockSpec((1, d), lambda i: (0, 0)),
        ],
        out_specs=pl.BlockSpec((blk, d), lambda i: (i, 0)),
        out_shape=jax.ShapeDtypeStruct((n_pad, d), jnp.float32),
    )


def kernel(x, edge_index, merge_W, merge_b):
    n, d = x.shape
    e_total = edge_index.shape[1]
    mult = R * NW * KB // 4
    n_pad = ((n + mult - 1) // mult) * mult

    row = edge_index[0]
    col = edge_index[1]

    agg_sum, agg_max = _seg_agg_kernel(n_pad, e_total, d)(row, col, x)

    x_pad = jnp.pad(x, ((0, n_pad - n), (0, 0)))
    w1 = merge_W[:d]
    w2 = merge_W[d:]
    out = _merge_kernel(n_pad, d)(
        x_pad, agg_sum, agg_max, w1, w2, merge_b.reshape(1, d))
    return out[:n]
